# baseline (speedup 1.0000x reference)
import jax
import jax.numpy as jnp
from jax import lax
from jax.experimental import pallas as pl
from jax.experimental.pallas import tpu as pltpu
from jax.experimental.pallas import tpu_sc as plsc

N = 10000
D = 128
CH = 128
NCHUNK = 160
NC = 2
NS = 16
EPAD = NS * NCHUNK * CH
HALF = 5120
DUM = 2048
ACC_ROWS = HALF + DUM
RPT = ACC_ROWS // NS
BLK = 1000
GRID = N // BLK


NBUF = 2


def _agg_body(xws, rowr, colr, zrows, out, rowv, colv,
              buf0, buf1, acc, g0, g1, s0, s1):
    c = lax.axis_index("c")
    s = lax.axis_index("s")
    pltpu.sync_copy(rowr.at[s], rowv)
    pltpu.sync_copy(colr.at[c, s], colv)
    pltpu.sync_copy(zrows, acc.at[pl.ds(s * RPT, RPT)])
    plsc.subcore_barrier()

    bufs = (buf0, buf1)
    gsems = (g0, g1)
    ssems = (s0, s1)
    for j in range(NBUF - 1):
        pltpu.async_copy(xws.at[rowv.at[j]], bufs[j], gsems[j])

    def body(i, carry):
        j0 = i * NBUF
        for b in range(NBUF):
            j = j0 + b
            bp = (b + NBUF - 1) % NBUF
            pltpu.make_async_copy(xws.at[rowv.at[j]], bufs[b], gsems[b]).wait()
            pltpu.async_copy(bufs[b], acc.at[colv.at[j]], ssems[b], add=True)

            @pl.when(j >= 1)
            def _():
                pltpu.make_async_copy(
                    bufs[bp], acc.at[colv.at[j - 1]], ssems[bp]).wait()

            @pl.when(j + NBUF - 1 < NCHUNK)
            def _():
                pltpu.async_copy(
                    xws.at[rowv.at[j + NBUF - 1]], bufs[bp], gsems[bp])

        return carry

    lax.fori_loop(0, NCHUNK // NBUF, body, 0)
    pltpu.make_async_copy(
        bufs[NBUF - 1], acc.at[colv.at[NCHUNK - 1]], ssems[NBUF - 1]).wait()
    plsc.subcore_barrier()
    base = s * RPT
    for off, nr in ((0, 128), (128, 128), (256, 128), (384, 64)):
        pltpu.sync_copy(acc.at[pl.ds(base + off, nr)], buf0.at[pl.ds(0, nr)])
        pltpu.sync_copy(buf0.at[pl.ds(0, nr)], out.at[c, pl.ds(base + off, nr)])


_sc_agg = pl.kernel(
    _agg_body,
    out_type=jax.ShapeDtypeStruct((NC, ACC_ROWS, D), jnp.float32),
    mesh=plsc.VectorSubcoreMesh(core_axis_name="c", subcore_axis_name="s"),
    scratch_types=[
        pltpu.VMEM((NCHUNK, CH), jnp.int32),
        pltpu.VMEM((NCHUNK, CH), jnp.int32),
        pltpu.VMEM((CH, D), jnp.float32),
        pltpu.VMEM((CH, D), jnp.float32),
        pltpu.VMEM_SHARED((ACC_ROWS, D), jnp.float32),
        pltpu.SemaphoreType.DMA,
        pltpu.SemaphoreType.DMA,
        pltpu.SemaphoreType.DMA,
        pltpu.SemaphoreType.DMA,
    ],
)


def _tc_step_body(p_ref, xws_ref, dis_ref, f_ref, flag_ref, b_ref, w_ref,
                  xn_ref, disn_ref, o_ref):
    first = flag_ref[0, 0] == 0.0
    psum = p_ref[...]
    deg = psum[:, 0:1] + 1.0
    dis_new = jnp.where(deg > 0, lax.rsqrt(deg), 0.0)
    dis = jnp.where(first, dis_new, dis_ref[...])
    o = dis * (psum + xws_ref[...]) + b_ref[...]
    o_ref[...] = o
    z = jnp.where(first, f_ref[...], jnp.maximum(o, 0.0))
    xn_ref[...] = jnp.dot(z, w_ref[...], preferred_element_type=jnp.float32) * dis
    disn_ref[...] = dis


def _tc_step(P, xws, dis2, feats, flag, b, Wn):
    return pl.pallas_call(
        _tc_step_body,
        grid=(GRID,),
        in_specs=[
            pl.BlockSpec((BLK, D), lambda i: (i, 0)),
            pl.BlockSpec((BLK, D), lambda i: (i, 0)),
            pl.BlockSpec((BLK, 1), lambda i: (i, 0)),
            pl.BlockSpec((BLK, D), lambda i: (i, 0)),
            pl.BlockSpec((1, 1), lambda i: (0, 0), memory_space=pltpu.SMEM),
            pl.BlockSpec((1, D), lambda i: (0, 0)),
            pl.BlockSpec((D, D), lambda i: (0, 0)),
        ],
        out_specs=[
            pl.BlockSpec((BLK, D), lambda i: (i, 0)),
            pl.BlockSpec((BLK, 1), lambda i: (i, 0)),
            pl.BlockSpec((BLK, D), lambda i: (i, 0)),
        ],
        out_shape=[
            jax.ShapeDtypeStruct((N, D), jnp.float32),
            jax.ShapeDtypeStruct((N, 1), jnp.float32),
            jax.ShapeDtypeStruct((N, D), jnp.float32),
        ],
    )(P, xws, dis2, feats, flag, b, Wn)


def kernel(features, edge_index, W1, b1, W2, b2, W3, b3):
    row = edge_index[0].astype(jnp.int32)
    col = edge_index[1].astype(jnp.int32)
    pad = EPAD - row.shape[0]
    rowp = jnp.pad(row, (0, pad)).reshape(NS, NCHUNK, CH)
    colx = jnp.pad(col, (0, pad), constant_values=-1)
    spread = HALF + (jnp.arange(EPAD, dtype=jnp.int32) % DUM)

    def remap(c):
        lc = colx - c * HALF
        inr = (lc >= 0) & (lc < HALF)
        return jnp.where(inr, lc, spread)

    colp = jnp.stack([remap(0), remap(1)]).reshape(NC, NS, NCHUNK, CH)
    zrows = jnp.zeros((RPT, D), jnp.float32)

    Wstack = jnp.stack([W1, W2, W3, jnp.eye(D, dtype=jnp.float32)])
    zb = jnp.zeros_like(b1)
    bstack = jnp.stack([zb, b1, b2, b3]).reshape(4, 1, D)
    flags = jnp.arange(4, dtype=jnp.float32).reshape(4, 1, 1)

    def step(carry, xs):
        xws, dis2, _ = carry
        flag, b_i, W_i = xs
        Pacc = _sc_agg(xws, rowp, colp, zrows)
        P = jnp.concatenate([Pacc[0, :HALF], Pacc[1, : N - HALF]])
        xn, disn, o = _tc_step(P, xws, dis2, features, flag, b_i, W_i)
        return (xn, disn, o), None

    init = (
        jnp.ones((N, D), jnp.float32),
        jnp.ones((N, 1), jnp.float32),
        jnp.zeros((N, D), jnp.float32),
    )
    (_, _, out), _ = lax.scan(step, init, (flags, bstack, Wstack))
    return out


# speedup vs baseline: 1.0398x; 1.0398x over previous
import jax
import jax.numpy as jnp
from jax import lax
from jax.experimental import pallas as pl
from jax.experimental.pallas import tpu as pltpu
from jax.experimental.pallas import tpu_sc as plsc

N = 10000
D = 128
CH = 128
NCHUNK = 160
NC = 2
NS = 16
EPAD = NS * NCHUNK * CH
HALF = 5120
DUM = 512
ACC_ROWS = HALF + DUM
RPT = ACC_ROWS // NS
BLK = 1000
GRID = N // BLK


def _agg_body(xws, rowr, colr, zrows, out, rowv, colv, buf0, buf1, acc, g0, g1):
    c = lax.axis_index("c")
    s = lax.axis_index("s")
    pltpu.sync_copy(rowr.at[s], rowv)
    pltpu.sync_copy(colr.at[c, s], colv)
    pltpu.sync_copy(zrows, acc.at[pl.ds(s * RPT, RPT)])
    plsc.subcore_barrier()

    bufs = (buf0, buf1)
    sems = (g0, g1)
    pltpu.async_copy(xws.at[rowv.at[0]], buf0, g0)
    pltpu.async_copy(xws.at[rowv.at[1]], buf1, g1)

    def body(i, carry):
        j0 = i * 2
        for b in range(2):
            j = j0 + b
            pltpu.make_async_copy(xws.at[rowv.at[j]], bufs[b], sems[b]).wait()
            pltpu.sync_copy(bufs[b], acc.at[colv.at[j]], add=True)

            @pl.when(j + 2 < NCHUNK)
            def _():
                pltpu.async_copy(xws.at[rowv.at[j + 2]], bufs[b], sems[b])

        return carry

    lax.fori_loop(0, NCHUNK // 2, body, 0)
    plsc.subcore_barrier()
    base = s * RPT
    for off, nr in ((0, 128), (128, 128), (256, 96)):
        pltpu.sync_copy(acc.at[pl.ds(base + off, nr)], buf0.at[pl.ds(0, nr)])
        pltpu.sync_copy(buf0.at[pl.ds(0, nr)], out.at[c, pl.ds(base + off, nr)])


_sc_agg = pl.kernel(
    _agg_body,
    out_type=jax.ShapeDtypeStruct((NC, ACC_ROWS, D), jnp.float32),
    mesh=plsc.VectorSubcoreMesh(core_axis_name="c", subcore_axis_name="s"),
    scratch_types=[
        pltpu.VMEM((NCHUNK, CH), jnp.int32),
        pltpu.VMEM((NCHUNK, CH), jnp.int32),
        pltpu.VMEM((CH, D), jnp.float32),
        pltpu.VMEM((CH, D), jnp.float32),
        pltpu.VMEM_SHARED((ACC_ROWS, D), jnp.float32),
        pltpu.SemaphoreType.DMA,
        pltpu.SemaphoreType.DMA,
    ],
)


def _tc_step_body(p_ref, xws_ref, dis_ref, f_ref, flag_ref, b_ref, w_ref,
                  xn_ref, disn_ref, o_ref):
    first = flag_ref[0, 0] == 0.0
    psum = p_ref[...]
    deg = psum[:, 0:1] + 1.0
    dis_new = jnp.where(deg > 0, lax.rsqrt(deg), 0.0)
    dis = jnp.where(first, dis_new, dis_ref[...])
    o = dis * (psum + xws_ref[...]) + b_ref[...]
    o_ref[...] = o
    z = jnp.where(first, f_ref[...], jnp.maximum(o, 0.0))
    xn_ref[...] = jnp.dot(z, w_ref[...], preferred_element_type=jnp.float32) * dis
    disn_ref[...] = dis


def _tc_step(P, xws, dis2, feats, flag, b, Wn):
    return pl.pallas_call(
        _tc_step_body,
        grid=(GRID,),
        in_specs=[
            pl.BlockSpec((BLK, D), lambda i: (i, 0)),
            pl.BlockSpec((BLK, D), lambda i: (i, 0)),
            pl.BlockSpec((BLK, 1), lambda i: (i, 0)),
            pl.BlockSpec((BLK, D), lambda i: (i, 0)),
            pl.BlockSpec((1, 1), lambda i: (0, 0), memory_space=pltpu.SMEM),
            pl.BlockSpec((1, D), lambda i: (0, 0)),
            pl.BlockSpec((D, D), lambda i: (0, 0)),
        ],
        out_specs=[
            pl.BlockSpec((BLK, D), lambda i: (i, 0)),
            pl.BlockSpec((BLK, 1), lambda i: (i, 0)),
            pl.BlockSpec((BLK, D), lambda i: (i, 0)),
        ],
        out_shape=[
            jax.ShapeDtypeStruct((N, D), jnp.float32),
            jax.ShapeDtypeStruct((N, 1), jnp.float32),
            jax.ShapeDtypeStruct((N, D), jnp.float32),
        ],
    )(P, xws, dis2, feats, flag, b, Wn)


def kernel(features, edge_index, W1, b1, W2, b2, W3, b3):
    row = edge_index[0].astype(jnp.int32)
    col = edge_index[1].astype(jnp.int32)
    pad = EPAD - row.shape[0]
    rowp = jnp.pad(row, (0, pad)).reshape(NS, NCHUNK, CH)
    colx = jnp.pad(col, (0, pad), constant_values=-1)
    spread = HALF + (jnp.arange(EPAD, dtype=jnp.int32) % DUM)

    def remap(c):
        lc = colx - c * HALF
        inr = (lc >= 0) & (lc < HALF)
        return jnp.where(inr, lc, spread)

    colp = jnp.stack([remap(0), remap(1)]).reshape(NC, NS, NCHUNK, CH)
    zrows = jnp.zeros((RPT, D), jnp.float32)

    Wstack = jnp.stack([W1, W2, W3, jnp.eye(D, dtype=jnp.float32)])
    zb = jnp.zeros_like(b1)
    bstack = jnp.stack([zb, b1, b2, b3]).reshape(4, 1, D)
    flags = jnp.arange(4, dtype=jnp.float32).reshape(4, 1, 1)

    def step(carry, xs):
        xws, dis2, _ = carry
        flag, b_i, W_i = xs
        Pacc = _sc_agg(xws, rowp, colp, zrows)
        P = jnp.concatenate([Pacc[0, :HALF], Pacc[1, : N - HALF]])
        xn, disn, o = _tc_step(P, xws, dis2, features, flag, b_i, W_i)
        return (xn, disn, o), None

    init = (
        jnp.ones((N, D), jnp.float32),
        jnp.ones((N, 1), jnp.float32),
        jnp.zeros((N, D), jnp.float32),
    )
    (_, _, out), _ = lax.scan(step, init, (flags, bstack, Wstack))
    return out
